# baseline (speedup 1.0000x reference)
import jax
import jax.numpy as jnp
from jax import lax
from jax.experimental import pallas as pl
from jax.experimental.pallas import tpu as pltpu
from jax.experimental.pallas import tpu_sc as plsc

N = 10000
E = 160000
D = 256
HALF = 128
NC = 2
NS = 16
CHUNK = 128
EPS = -(-E // NS)
NCHUNK = 4 * (-(-EPS // (4 * CHUNK)))
EPAD = NS * NCHUNK * CHUNK
NFULL = N // CHUNK
NREM = N - NFULL * CHUNK
LANES = 16
IDEPTH = 4
NH = -(-N // 128)
EC = 2000
NECH = E // EC


def _spmm_body(x_hbm, eidx_hbm, eadj_hbm, out_hbm,
               ebuf, abuf, rows, rows2, acc, si0, si1, si2, si3, sg0, sg1):
    c = lax.axis_index("c")
    s = lax.axis_index("s")
    bufs = (rows, rows2)
    isems = (si0, si1, si2, si3)
    gsems = (sg0, sg1)

    zero = jnp.zeros((LANES,), jnp.float32)

    def zb(e, _):
        for j in range(HALF // LANES):
            rows[e, pl.ds(j * LANES, LANES)] = zero
        return 0

    lax.fori_loop(0, CHUNK, zb, 0)
    for t in range(-(-(NFULL + 1) // NS)):
        cid = t * NS + s

        @pl.when(cid < NFULL)
        def _():
            pltpu.sync_copy(rows, acc.at[pl.ds(cid * CHUNK, CHUNK)])

        @pl.when(cid == NFULL)
        def _():
            pltpu.sync_copy(rows.at[pl.ds(0, NREM)],
                            acc.at[pl.ds(NFULL * CHUNK, NREM)])
    plsc.subcore_barrier()

    def stage(k, p):
        pltpu.async_copy(eidx_hbm.at[s].at[k], ebuf.at[p], isems[p])
        pltpu.async_copy(eadj_hbm.at[s].at[k], abuf.at[p], isems[p])

    def wait_i(k, p):
        pltpu.make_async_copy(eidx_hbm.at[s].at[k], ebuf.at[p],
                              isems[p]).wait()
        pltpu.make_async_copy(eadj_hbm.at[s].at[k], abuf.at[p],
                              isems[p]).wait()

    def gath(k, p, b):
        pltpu.async_copy(x_hbm.at[c].at[ebuf.at[p, 0]], bufs[b], gsems[b])

    def proc(k, p, b):
        buf = bufs[b]
        pltpu.make_async_copy(x_hbm.at[c].at[ebuf.at[p, 0]], buf,
                              gsems[b]).wait()

        def scale(g, _):
            a16 = abuf[p, pl.ds(g * LANES, LANES)]
            for l in range(LANES):
                e = g * LANES + l
                for j in range(HALF // LANES):
                    sl = pl.ds(j * LANES, LANES)
                    buf[e, sl] = buf[e, sl] * a16[l]
            return 0

        lax.fori_loop(0, CHUNK // LANES, scale, 0)
        pltpu.sync_copy(buf, acc.at[ebuf.at[p, 1]], add=True)

    for k in range(IDEPTH):
        stage(k, k)
    wait_i(0, 0)
    gath(0, 0, 0)

    def chunk_body(i, _):
        k0 = IDEPTH * i
        for j in range(IDEPTH):
            k = k0 + j
            pn = (j + 1) % IDEPTH

            @pl.when(k + 1 < NCHUNK)
            def _():
                wait_i(k + 1, pn)
                gath(k + 1, pn, (j + 1) % 2)

            proc(k, j, j % 2)

            @pl.when(k + IDEPTH < NCHUNK)
            def _():
                stage(k + IDEPTH, j)

        return 0

    lax.fori_loop(0, NCHUNK // IDEPTH, chunk_body, 0)
    plsc.subcore_barrier()

    for t in range(-(-(NFULL + 1) // NS)):
        cid = t * NS + s

        @pl.when(cid < NFULL)
        def _():
            pltpu.sync_copy(acc.at[pl.ds(cid * CHUNK, CHUNK)],
                            out_hbm.at[c].at[pl.ds(cid * CHUNK, CHUNK)])

        @pl.when(cid == NFULL)
        def _():
            pltpu.sync_copy(acc.at[pl.ds(NFULL * CHUNK, NREM)],
                            out_hbm.at[c].at[pl.ds(NFULL * CHUNK, NREM)])


_spmm_call = pl.kernel(
    _spmm_body,
    out_type=jax.ShapeDtypeStruct((NC, N, HALF), jnp.float32),
    mesh=plsc.VectorSubcoreMesh(core_axis_name="c", subcore_axis_name="s"),
    scratch_types=[
        pltpu.VMEM((IDEPTH, 2, CHUNK), jnp.int32),
        pltpu.VMEM((IDEPTH, CHUNK), jnp.float32),
        pltpu.VMEM((CHUNK, HALF), jnp.float32),
        pltpu.VMEM((CHUNK, HALF), jnp.float32),
        pltpu.VMEM_SHARED((N, HALF), jnp.float32),
        pltpu.SemaphoreType.DMA,
        pltpu.SemaphoreType.DMA,
        pltpu.SemaphoreType.DMA,
        pltpu.SemaphoreType.DMA,
        pltpu.SemaphoreType.DMA,
        pltpu.SemaphoreType.DMA,
    ],
)


def _deg_body(row_ref, col_ref, adj_ref, deg_ref, deg2_ref):
    iota_l = lax.broadcasted_iota(jnp.int32, (EC, 128), 1)
    iota_h = lax.broadcasted_iota(jnp.int32, (EC, NH + 1), 1)
    cdims = (((0,), (0,)), ((), ()))

    def stage1(i, dm):
        r = row_ref[i][:, None]
        a = adj_ref[i][:, None]
        ohl = jnp.where(lax.rem(r, 128) == iota_l, a, 0.0)
        ohh = (r // 128 == iota_h).astype(jnp.float32)
        return dm + lax.dot_general(ohh, ohl, cdims,
                                    preferred_element_type=jnp.float32)

    dm = lax.fori_loop(0, NECH, stage1,
                       jnp.zeros((NH + 1, 128), jnp.float32))
    deg_ref[...] = dm

    def stage2(i, dm2):
        cq = col_ref[i][:, None]
        t = lax.dot_general((cq // 128 == iota_h).astype(jnp.float32), dm,
                            (((1,), (0,)), ((), ())),
                            preferred_element_type=jnp.float32)
        g = jnp.sum(jnp.where(lax.rem(cq, 128) == iota_l, t, 0.0), axis=1)
        z = (g * adj_ref[i])[:, None]
        r = row_ref[i][:, None]
        ohl = jnp.where(lax.rem(r, 128) == iota_l, z, 0.0)
        ohh = (r // 128 == iota_h).astype(jnp.float32)
        return dm2 + lax.dot_general(ohh, ohl, cdims,
                                     preferred_element_type=jnp.float32)

    deg2_ref[...] = lax.fori_loop(0, NECH, stage2,
                                  jnp.zeros((NH + 1, 128), jnp.float32))


_degchain = pl.pallas_call(
    _deg_body,
    out_shape=(jax.ShapeDtypeStruct((NH + 1, 128), jnp.float32),
               jax.ShapeDtypeStruct((NH + 1, 128), jnp.float32)),
)


BN = 2000


def _prep_body(w_ref, b_ref, m1_ref, m2_ref, c_ref):
    w0, w1, w2 = w_ref[0], w_ref[1], w_ref[2]
    b0, b1, b2 = b_ref[0:1], b_ref[1:2], b_ref[2:3]
    m1 = jnp.dot(w0, w1, preferred_element_type=jnp.float32)
    m2 = jnp.dot(m1, w2, preferred_element_type=jnp.float32)
    t01 = jnp.dot(b0, w1, preferred_element_type=jnp.float32)
    m1_ref[...] = m1
    m2_ref[...] = m2
    c_ref[0:1] = b0 + b1 + b2
    c_ref[1:2] = t01 + jnp.dot(b1, w2, preferred_element_type=jnp.float32)
    c_ref[2:3] = jnp.dot(t01, w2, preferred_element_type=jnp.float32)


_prep = pl.pallas_call(
    _prep_body,
    out_shape=(jax.ShapeDtypeStruct((D, D), jnp.float32),
               jax.ShapeDtypeStruct((D, D), jnp.float32),
               jax.ShapeDtypeStruct((3, D), jnp.float32)),
)


def _comb_body(f_ref, s1_ref, s2_ref, s3_ref, d1_ref, d2_ref,
               m0m_ref, m1m_ref, m2m_ref, cm_ref,
               m0s_ref, m1s_ref, m2s_ref, cs_ref, om_ref, os_ref):
    f = f_ref[...]
    d1 = d1_ref[...]
    d2 = d2_ref[...]
    for ms, c_ref, o_ref in (
            ((m0m_ref, m1m_ref, m2m_ref), cm_ref, om_ref),
            ((m0s_ref, m1s_ref, m2s_ref), cs_ref, os_ref)):
        acc = f
        for s_ref, m_ref in zip((s1_ref, s2_ref, s3_ref), ms):
            m = m_ref[...]
            acc = acc + jnp.dot(s_ref[0], m[:HALF],
                                preferred_element_type=jnp.float32)
            acc = acc + jnp.dot(s_ref[1], m[HALF:],
                                preferred_element_type=jnp.float32)
        acc = acc + c_ref[0:1]
        acc = acc + d1 * c_ref[1:2]
        acc = acc + d2 * c_ref[2:3]
        o_ref[...] = acc * 0.25


_wblk = pl.BlockSpec((D, D), lambda i: (0, 0))
_cblk = pl.BlockSpec((3, D), lambda i: (0, 0))
_combine = pl.pallas_call(
    _comb_body,
    grid=(N // BN,),
    in_specs=[
        pl.BlockSpec((BN, D), lambda i: (i, 0)),
        pl.BlockSpec((NC, BN, HALF), lambda i: (0, i, 0)),
        pl.BlockSpec((NC, BN, HALF), lambda i: (0, i, 0)),
        pl.BlockSpec((NC, BN, HALF), lambda i: (0, i, 0)),
        pl.BlockSpec((BN, 1), lambda i: (i, 0)),
        pl.BlockSpec((BN, 1), lambda i: (i, 0)),
        _wblk, _wblk, _wblk, _cblk,
        _wblk, _wblk, _wblk, _cblk,
    ],
    out_specs=(pl.BlockSpec((BN, D), lambda i: (i, 0)),
               pl.BlockSpec((BN, D), lambda i: (i, 0))),
    out_shape=(jax.ShapeDtypeStruct((N, D), jnp.float32),
               jax.ShapeDtypeStruct((N, D), jnp.float32)),
)


def kernel(fea, edge_index, adj_values, mu_W, mu_b, sigma_W, sigma_b):
    row = edge_index[0].astype(jnp.int32)
    col = edge_index[1].astype(jnp.int32)
    pad = EPAD - E
    eidx = jnp.stack([jnp.pad(col, (0, pad)), jnp.pad(row, (0, pad))])
    eidx = eidx.reshape(2, NS, NCHUNK, CHUNK).transpose(1, 2, 0, 3)
    eadj = jnp.pad(adj_values, (0, pad)).reshape(NS, NCHUNK, CHUNK)
    fea2 = fea.reshape(N, NC, HALF).transpose(1, 0, 2)

    dg1, dg2 = _degchain(row.reshape(NECH, EC), col.reshape(NECH, EC),
                         adj_values.reshape(NECH, EC))
    d1 = dg1.reshape(-1, 1)[:N]
    d2 = dg2.reshape(-1, 1)[:N]

    s1 = _spmm_call(fea2, eidx, eadj)
    s2 = _spmm_call(s1, eidx, eadj)
    s3 = _spmm_call(s2, eidx, eadj)

    m1m, m2m, cm = _prep(mu_W, mu_b)
    m1s, m2s, cs = _prep(sigma_W, sigma_b)
    mu, sigma = _combine(fea, s1, s2, s3, d1, d2,
                         mu_W[0], m1m, m2m, cm,
                         sigma_W[0], m1s, m2s, cs)

    part = (row >= N // 2).astype(jnp.int32)
    c0 = jnp.cumsum(1 - part)
    c1 = jnp.cumsum(part)
    pos = jnp.where(part == 0, c0 - 1, EPAD + c1 - 1)
    dcol = jnp.zeros((2 * EPAD,), jnp.int32).at[pos].set(col, unique_indices=True)
    drow = jnp.zeros((2 * EPAD,), jnp.int32).at[pos].set(row, unique_indices=True)
    dadj = jnp.zeros((2 * EPAD,), jnp.float32).at[pos].set(adj_values, unique_indices=True)
    probe_t = jnp.minimum(dadj[5] + dcol[7].astype(jnp.float32) + drow[9].astype(jnp.float32), 0.0)
    mu = mu + probe_t
    return mu, sigma


# speedup vs baseline: 2.7456x; 2.7456x over previous
import jax
import jax.numpy as jnp
from jax import lax
from jax.experimental import pallas as pl
from jax.experimental.pallas import tpu as pltpu
from jax.experimental.pallas import tpu_sc as plsc

N = 10000
E = 160000
D = 256
HALF = 128
NC = 2
NS = 16
CHUNK = 128
EPS = -(-E // NS)
NCHUNK = 4 * (-(-EPS // (4 * CHUNK)))
EPAD = NS * NCHUNK * CHUNK
NFULL = N // CHUNK
NREM = N - NFULL * CHUNK
LANES = 16
IDEPTH = 4
NH = -(-N // 128)
EC = 2000
NECH = E // EC


def _spmm_body(x_hbm, eidx_hbm, eadj_hbm, out_hbm,
               ebuf, abuf, rows, rows2, acc, si0, si1, si2, si3, sg0, sg1):
    c = lax.axis_index("c")
    s = lax.axis_index("s")
    bufs = (rows, rows2)
    isems = (si0, si1, si2, si3)
    gsems = (sg0, sg1)

    zero = jnp.zeros((LANES,), jnp.float32)

    def zb(e, _):
        for j in range(HALF // LANES):
            rows[e, pl.ds(j * LANES, LANES)] = zero
        return 0

    lax.fori_loop(0, CHUNK, zb, 0)
    for t in range(-(-(NFULL + 1) // NS)):
        cid = t * NS + s

        @pl.when(cid < NFULL)
        def _():
            pltpu.sync_copy(rows, acc.at[pl.ds(cid * CHUNK, CHUNK)])

        @pl.when(cid == NFULL)
        def _():
            pltpu.sync_copy(rows.at[pl.ds(0, NREM)],
                            acc.at[pl.ds(NFULL * CHUNK, NREM)])
    plsc.subcore_barrier()

    def stage(k, p):
        pltpu.async_copy(eidx_hbm.at[s].at[k], ebuf.at[p], isems[p])
        pltpu.async_copy(eadj_hbm.at[s].at[k], abuf.at[p], isems[p])

    def wait_i(k, p):
        pltpu.make_async_copy(eidx_hbm.at[s].at[k], ebuf.at[p],
                              isems[p]).wait()
        pltpu.make_async_copy(eadj_hbm.at[s].at[k], abuf.at[p],
                              isems[p]).wait()

    def gath(k, p, b):
        pltpu.async_copy(x_hbm.at[c].at[ebuf.at[p, 0]], bufs[b], gsems[b])

    def proc(k, p, b):
        buf = bufs[b]
        pltpu.make_async_copy(x_hbm.at[c].at[ebuf.at[p, 0]], buf,
                              gsems[b]).wait()

        def scale(g, _):
            a16 = abuf[p, pl.ds(g * LANES, LANES)]
            for l in range(LANES):
                e = g * LANES + l
                for j in range(HALF // LANES):
                    sl = pl.ds(j * LANES, LANES)
                    buf[e, sl] = buf[e, sl] * a16[l]
            return 0

        lax.fori_loop(0, CHUNK // LANES, scale, 0)
        pltpu.sync_copy(buf, acc.at[ebuf.at[p, 1]], add=True)

    for k in range(IDEPTH):
        stage(k, k)
    wait_i(0, 0)
    gath(0, 0, 0)

    def chunk_body(i, _):
        k0 = IDEPTH * i
        for j in range(IDEPTH):
            k = k0 + j
            pn = (j + 1) % IDEPTH

            @pl.when(k + 1 < NCHUNK)
            def _():
                wait_i(k + 1, pn)
                gath(k + 1, pn, (j + 1) % 2)

            proc(k, j, j % 2)

            @pl.when(k + IDEPTH < NCHUNK)
            def _():
                stage(k + IDEPTH, j)

        return 0

    lax.fori_loop(0, NCHUNK // IDEPTH, chunk_body, 0)
    plsc.subcore_barrier()

    for t in range(-(-(NFULL + 1) // NS)):
        cid = t * NS + s

        @pl.when(cid < NFULL)
        def _():
            pltpu.sync_copy(acc.at[pl.ds(cid * CHUNK, CHUNK)],
                            out_hbm.at[c].at[pl.ds(cid * CHUNK, CHUNK)])

        @pl.when(cid == NFULL)
        def _():
            pltpu.sync_copy(acc.at[pl.ds(NFULL * CHUNK, NREM)],
                            out_hbm.at[c].at[pl.ds(NFULL * CHUNK, NREM)])


_spmm_call = pl.kernel(
    _spmm_body,
    out_type=jax.ShapeDtypeStruct((NC, N, HALF), jnp.float32),
    mesh=plsc.VectorSubcoreMesh(core_axis_name="c", subcore_axis_name="s"),
    scratch_types=[
        pltpu.VMEM((IDEPTH, 2, CHUNK), jnp.int32),
        pltpu.VMEM((IDEPTH, CHUNK), jnp.float32),
        pltpu.VMEM((CHUNK, HALF), jnp.float32),
        pltpu.VMEM((CHUNK, HALF), jnp.float32),
        pltpu.VMEM_SHARED((N, HALF), jnp.float32),
        pltpu.SemaphoreType.DMA,
        pltpu.SemaphoreType.DMA,
        pltpu.SemaphoreType.DMA,
        pltpu.SemaphoreType.DMA,
        pltpu.SemaphoreType.DMA,
        pltpu.SemaphoreType.DMA,
    ],
)


def _deg_body(row_ref, col_ref, adj_ref, deg_ref, deg2_ref):
    iota_l = lax.broadcasted_iota(jnp.int32, (EC, 128), 1)
    iota_h = lax.broadcasted_iota(jnp.int32, (EC, NH + 1), 1)
    cdims = (((0,), (0,)), ((), ()))

    def stage1(i, dm):
        r = row_ref[i][:, None]
        a = adj_ref[i][:, None]
        ohl = jnp.where(lax.rem(r, 128) == iota_l, a, 0.0)
        ohh = (r // 128 == iota_h).astype(jnp.float32)
        return dm + lax.dot_general(ohh, ohl, cdims,
                                    preferred_element_type=jnp.float32)

    dm = lax.fori_loop(0, NECH, stage1,
                       jnp.zeros((NH + 1, 128), jnp.float32))
    deg_ref[...] = dm

    def stage2(i, dm2):
        cq = col_ref[i][:, None]
        t = lax.dot_general((cq // 128 == iota_h).astype(jnp.float32), dm,
                            (((1,), (0,)), ((), ())),
                            preferred_element_type=jnp.float32)
        g = jnp.sum(jnp.where(lax.rem(cq, 128) == iota_l, t, 0.0), axis=1)
        z = (g * adj_ref[i])[:, None]
        r = row_ref[i][:, None]
        ohl = jnp.where(lax.rem(r, 128) == iota_l, z, 0.0)
        ohh = (r // 128 == iota_h).astype(jnp.float32)
        return dm2 + lax.dot_general(ohh, ohl, cdims,
                                     preferred_element_type=jnp.float32)

    deg2_ref[...] = lax.fori_loop(0, NECH, stage2,
                                  jnp.zeros((NH + 1, 128), jnp.float32))


_degchain = pl.pallas_call(
    _deg_body,
    out_shape=(jax.ShapeDtypeStruct((NH + 1, 128), jnp.float32),
               jax.ShapeDtypeStruct((NH + 1, 128), jnp.float32)),
)


BN = 2000


def _prep_body(w_ref, b_ref, m1_ref, m2_ref, c_ref):
    w0, w1, w2 = w_ref[0], w_ref[1], w_ref[2]
    b0, b1, b2 = b_ref[0:1], b_ref[1:2], b_ref[2:3]
    m1 = jnp.dot(w0, w1, preferred_element_type=jnp.float32)
    m2 = jnp.dot(m1, w2, preferred_element_type=jnp.float32)
    t01 = jnp.dot(b0, w1, preferred_element_type=jnp.float32)
    m1_ref[...] = m1
    m2_ref[...] = m2
    c_ref[0:1] = b0 + b1 + b2
    c_ref[1:2] = t01 + jnp.dot(b1, w2, preferred_element_type=jnp.float32)
    c_ref[2:3] = jnp.dot(t01, w2, preferred_element_type=jnp.float32)


_prep = pl.pallas_call(
    _prep_body,
    out_shape=(jax.ShapeDtypeStruct((D, D), jnp.float32),
               jax.ShapeDtypeStruct((D, D), jnp.float32),
               jax.ShapeDtypeStruct((3, D), jnp.float32)),
)


def _comb_body(f_ref, s1_ref, s2_ref, s3_ref, d1_ref, d2_ref,
               m0m_ref, m1m_ref, m2m_ref, cm_ref,
               m0s_ref, m1s_ref, m2s_ref, cs_ref, om_ref, os_ref):
    f = f_ref[...]
    d1 = d1_ref[...]
    d2 = d2_ref[...]
    for ms, c_ref, o_ref in (
            ((m0m_ref, m1m_ref, m2m_ref), cm_ref, om_ref),
            ((m0s_ref, m1s_ref, m2s_ref), cs_ref, os_ref)):
        acc = f
        for s_ref, m_ref in zip((s1_ref, s2_ref, s3_ref), ms):
            m = m_ref[...]
            acc = acc + jnp.dot(s_ref[0], m[:HALF],
                                preferred_element_type=jnp.float32)
            acc = acc + jnp.dot(s_ref[1], m[HALF:],
                                preferred_element_type=jnp.float32)
        acc = acc + c_ref[0:1]
        acc = acc + d1 * c_ref[1:2]
        acc = acc + d2 * c_ref[2:3]
        o_ref[...] = acc * 0.25


_wblk = pl.BlockSpec((D, D), lambda i: (0, 0))
_cblk = pl.BlockSpec((3, D), lambda i: (0, 0))
_combine = pl.pallas_call(
    _comb_body,
    grid=(N // BN,),
    in_specs=[
        pl.BlockSpec((BN, D), lambda i: (i, 0)),
        pl.BlockSpec((NC, BN, HALF), lambda i: (0, i, 0)),
        pl.BlockSpec((NC, BN, HALF), lambda i: (0, i, 0)),
        pl.BlockSpec((NC, BN, HALF), lambda i: (0, i, 0)),
        pl.BlockSpec((BN, 1), lambda i: (i, 0)),
        pl.BlockSpec((BN, 1), lambda i: (i, 0)),
        _wblk, _wblk, _wblk, _cblk,
        _wblk, _wblk, _wblk, _cblk,
    ],
    out_specs=(pl.BlockSpec((BN, D), lambda i: (i, 0)),
               pl.BlockSpec((BN, D), lambda i: (i, 0))),
    out_shape=(jax.ShapeDtypeStruct((N, D), jnp.float32),
               jax.ShapeDtypeStruct((N, D), jnp.float32)),
)


def kernel(fea, edge_index, adj_values, mu_W, mu_b, sigma_W, sigma_b):
    row = edge_index[0].astype(jnp.int32)
    col = edge_index[1].astype(jnp.int32)
    pad = EPAD - E
    eidx = jnp.stack([jnp.pad(col, (0, pad)), jnp.pad(row, (0, pad))])
    eidx = eidx.reshape(2, NS, NCHUNK, CHUNK).transpose(1, 2, 0, 3)
    eadj = jnp.pad(adj_values, (0, pad)).reshape(NS, NCHUNK, CHUNK)
    fea2 = fea.reshape(N, NC, HALF).transpose(1, 0, 2)

    dg1, dg2 = _degchain(row.reshape(NECH, EC), col.reshape(NECH, EC),
                         adj_values.reshape(NECH, EC))
    d1 = dg1.reshape(-1, 1)[:N]
    d2 = dg2.reshape(-1, 1)[:N]

    s1 = _spmm_call(fea2, eidx, eadj)
    s2 = _spmm_call(s1, eidx, eadj)
    s3 = _spmm_call(s2, eidx, eadj)

    m1m, m2m, cm = _prep(mu_W, mu_b)
    m1s, m2s, cs = _prep(sigma_W, sigma_b)
    mu, sigma = _combine(fea, s1, s2, s3, d1, d2,
                         mu_W[0], m1m, m2m, cm,
                         sigma_W[0], m1s, m2s, cs)
    return mu, sigma
